# baseline (speedup 1.0000x reference)
import functools

import jax
import jax.numpy as jnp
from jax import lax
from jax.experimental import pallas as pl
from jax.experimental.pallas import tpu as pltpu
from jax.experimental.pallas import tpu_sc as plsc

_CELL_N = 16384
_CHUNK = 8192
_CHUNK_SHIFT = 13
_NW = 32
_OUT_BLOCKS = 8


def _sc_expression(idx_p, gene_ix, w_flat, b_flat, seg_total):
    g_n = gene_ix.shape[0]
    seg_per_w = seg_total // _NW
    rows_per_w = seg_per_w // g_n
    blk_rows = rows_per_w // _OUT_BLOCKS
    blk_seg = seg_per_w // _OUT_BLOCKS
    n = idx_p.shape[0]
    n_chunks = -(-n // _CHUNK)
    tail = n - (n_chunks - 1) * _CHUNK
    n_rounds = -(-(n - 1).bit_length() // 4)

    mesh = plsc.VectorSubcoreMesh(core_axis_name="c", subcore_axis_name="s")
    cp = pltpu.CompilerParams(needs_layout_passes=False)

    @functools.partial(
        pl.kernel,
        out_type=jax.ShapeDtypeStruct((seg_total,), jnp.float32),
        mesh=mesh,
        compiler_params=cp,
        scratch_types=[
            pltpu.VMEM((seg_per_w,), jnp.float32),
            pltpu.VMEM((_CHUNK,), jnp.int32),
            pltpu.VMEM((_CHUNK,), jnp.int32),
            pltpu.VMEM((_CHUNK,), jnp.int32),
            pltpu.VMEM((_CHUNK,), jnp.int32),
            pltpu.VMEM((16,), jnp.int32),
            pltpu.VMEM((16,), jnp.int32),
            pltpu.VMEM((16,), jnp.int32),
            pltpu.VMEM((16,), jnp.int32),
            pltpu.VMEM((g_n,), jnp.int32),
            pltpu.VMEM((g_n,), jnp.float32),
            pltpu.VMEM((g_n,), jnp.float32),
            pltpu.VMEM((g_n,), jnp.float32),
            pltpu.VMEM((g_n,), jnp.float32),
            pltpu.SemaphoreType.DMA,
            pltpu.SemaphoreType.DMA,
            pltpu.SemaphoreType.DMA,
            pltpu.SemaphoreType.DMA,
            pltpu.SemaphoreType.DMA,
            pltpu.SemaphoreType.DMA,
        ],
    )
    def k(idx_hbm, gix_hbm, w_hbm, b_hbm, out_hbm,
          counts_v, buf_a, buf_b, buf_c, buf_e,
          probe_a, probe_b, gath_a, gath_b,
          gi_v, wt_v, bt_v, wg_v, bg_v,
          sem_a, sem_b, sem_c, sem_e, sem_t, sem_o):
        wid = lax.axis_index("s") * 2 + lax.axis_index("c")
        seg_lo = wid * seg_per_w
        seg_hi = seg_lo + seg_per_w

        zeros16 = jnp.zeros((16,), jnp.float32)
        ones16 = jnp.ones((16,), jnp.float32)
        iota16 = lax.iota(jnp.int32, 16)

        pltpu.make_async_copy(gix_hbm, gi_v, sem_t).start()
        pltpu.make_async_copy(w_hbm, wt_v, sem_t).start()
        pltpu.make_async_copy(b_hbm, bt_v, sem_t).start()

        sc_search = jax.named_scope("span_search")
        sc_search.__enter__()
        la = jnp.int32(0)
        lb = jnp.int32(0)
        stage = 16 ** 3
        n_indirect = n_rounds - 3 if n >= stage else n_rounds
        for r in range(n_rounds - 1, n_rounds - 1 - n_indirect, -1):
            step = 16 ** r
            probe_a[...] = jnp.minimum(la + (iota16 + 1) * step - 1, n - 1)
            probe_b[...] = jnp.minimum(lb + (iota16 + 1) * step - 1, n - 1)
            ca = pltpu.async_copy(idx_hbm.at[probe_a], gath_a, sem_a)
            cb = pltpu.async_copy(idx_hbm.at[probe_b], gath_b, sem_b)
            ca.wait()
            cb.wait()
            cnt_a = plsc.all_reduce_population_count(gath_a[...] < seg_lo)
            cnt_b = plsc.all_reduce_population_count(gath_b[...] < seg_hi)
            la = jnp.minimum(la + jnp.max(cnt_a) * step, n)
            lb = jnp.minimum(lb + jnp.max(cnt_b) * step, n)
        if n >= stage:
            base_a = pl.multiple_of(jnp.minimum(la, n - stage), 8)
            base_b = pl.multiple_of(jnp.minimum(lb, n - stage), 8)
            ca = pltpu.async_copy(
                idx_hbm.at[pl.ds(base_a, stage)],
                buf_a.at[pl.ds(0, stage)], sem_a)
            cb = pltpu.async_copy(
                idx_hbm.at[pl.ds(base_b, stage)],
                buf_b.at[pl.ds(0, stage)], sem_b)
            ca.wait()
            cb.wait()
            for r in range(2, -1, -1):
                step = 16 ** r
                pa = jnp.minimum(la + (iota16 + 1) * step - 1, n - 1) - base_a
                pb = jnp.minimum(lb + (iota16 + 1) * step - 1, n - 1) - base_b
                va = plsc.load_gather(buf_a, [pa])
                vb = plsc.load_gather(buf_b, [pb])
                cnt_a = plsc.all_reduce_population_count(va < seg_lo)
                cnt_b = plsc.all_reduce_population_count(vb < seg_hi)
                la = jnp.minimum(la + jnp.max(cnt_a) * step, n)
                lb = jnp.minimum(lb + jnp.max(cnt_b) * step, n)
        frag_lo, frag_hi = la, lb
        sc_search.__exit__(None, None, None)
        sc_zero = jax.named_scope("zero_and_prefetch")
        sc_zero.__enter__()

        c_lo = frag_lo >> _CHUNK_SHIFT
        c_hi = (frag_hi + (_CHUNK - 1)) >> _CHUNK_SHIFT
        nch = c_hi - c_lo
        mid_hi = jnp.maximum(nch - 1, 1)

        def chunk_base(c):
            return pl.multiple_of((c_lo + c) * _CHUNK, _CHUNK)

        def bchunk_copy(glob, base, buf, sem):
            if tail != _CHUNK:
                @pl.when(glob == n_chunks - 1)
                def _():
                    pltpu.make_async_copy(
                        idx_hbm.at[pl.ds(base, tail)],
                        buf.at[pl.ds(0, tail)], sem).start()

                @pl.when(glob != n_chunks - 1)
                def _():
                    pltpu.make_async_copy(
                        idx_hbm.at[pl.ds(base, _CHUNK)], buf, sem).start()
            else:
                pltpu.make_async_copy(
                    idx_hbm.at[pl.ds(base, _CHUNK)], buf, sem).start()

        def bchunk_wait_process(glob, base, buf, sem):
            def masked_subchunks(n_sub):
                @plsc.parallel_loop(0, n_sub, unroll=8)
                def _(j):
                    v = buf[pl.ds(j * 16, 16)]
                    m = (v >= seg_lo) & (v < seg_hi)
                    lv = jnp.where(m, v - seg_lo, 0)
                    plsc.addupdate_scatter(counts_v, [lv], ones16, mask=m)

            if tail != _CHUNK:
                @pl.when(glob == n_chunks - 1)
                def _():
                    pltpu.make_async_copy(
                        idx_hbm.at[pl.ds(base, tail)],
                        buf.at[pl.ds(0, tail)], sem).wait()
                    masked_subchunks(tail // 16)

                @pl.when(glob != n_chunks - 1)
                def _():
                    pltpu.make_async_copy(
                        idx_hbm.at[pl.ds(base, _CHUNK)], buf, sem).wait()
                    masked_subchunks(_CHUNK // 16)
            else:
                pltpu.make_async_copy(
                    idx_hbm.at[pl.ds(base, _CHUNK)], buf, sem).wait()
                masked_subchunks(_CHUNK // 16)

        def start(c, buf, sem):
            pltpu.make_async_copy(
                idx_hbm.at[pl.ds(chunk_base(c), _CHUNK)], buf, sem).start()

        def wait(c, buf, sem):
            pltpu.make_async_copy(
                idx_hbm.at[pl.ds(chunk_base(c), _CHUNK)], buf, sem).wait()

        @pl.when(mid_hi > 1)
        def _():
            start(1, buf_a, sem_a)

        @pl.when(nch >= 1)
        def _():
            bchunk_copy(c_lo, chunk_base(0), buf_e, sem_e)

        @pl.when(nch >= 2)
        def _():
            bchunk_copy(c_hi - 1, chunk_base(nch - 1), buf_c, sem_c)

        @plsc.parallel_loop(0, seg_per_w // 16, unroll=8)
        def _(j):
            counts_v[pl.ds(j * 16, 16)] = zeros16

        sc_zero.__exit__(None, None, None)
        sc_hist = jax.named_scope("histogram")
        sc_hist.__enter__()
        @pl.when(nch >= 1)
        def _():
            bchunk_wait_process(c_lo, chunk_base(0), buf_e, sem_e)

        def process(buf):
            @plsc.parallel_loop(0, _CHUNK // 16, unroll=16)
            def _(j):
                v = buf[pl.ds(j * 16, 16)]
                plsc.addupdate_scatter(counts_v, [v - seg_lo], ones16)

        @pl.loop(1, mid_hi, step=2)
        def _(c):
            @pl.when(c + 1 < mid_hi)
            def _():
                start(c + 1, buf_b, sem_b)
            wait(c, buf_a, sem_a)
            process(buf_a)

            @pl.when(c + 1 < mid_hi)
            def _():
                @pl.when(c + 2 < mid_hi)
                def _():
                    start(c + 2, buf_a, sem_a)
                wait(c + 1, buf_b, sem_b)
                process(buf_b)

        @pl.when(nch >= 2)
        def _():
            bchunk_wait_process(c_hi - 1, chunk_base(nch - 1), buf_c, sem_c)

        sc_hist.__exit__(None, None, None)
        sc_read = jax.named_scope("readout")
        sc_read.__enter__()
        pltpu.make_async_copy(gix_hbm, gi_v, sem_t).wait()
        pltpu.make_async_copy(w_hbm, wt_v, sem_t).wait()
        pltpu.make_async_copy(b_hbm, bt_v, sem_t).wait()
        for t in range(g_n // 16):
            g16 = gi_v[pl.ds(t * 16, 16)]
            wg_v[pl.ds(t * 16, 16)] = plsc.load_gather(wt_v, [g16])
            bg_v[pl.ds(t * 16, 16)] = plsc.load_gather(bt_v, [g16])

        for blk in range(_OUT_BLOCKS):
            @plsc.parallel_loop(blk * blk_rows, (blk + 1) * blk_rows,
                                unroll=8)
            def _(r):
                row = r * g_n
                for t in range(g_n // 16):
                    off = row + t * 16
                    counts_v[pl.ds(off, 16)] = (
                        counts_v[pl.ds(off, 16)] * wg_v[pl.ds(t * 16, 16)]
                        + bg_v[pl.ds(t * 16, 16)])
            pltpu.make_async_copy(
                counts_v.at[pl.ds(blk * blk_seg, blk_seg)],
                out_hbm.at[pl.ds(
                    pl.multiple_of(seg_lo + blk * blk_seg, blk_seg),
                    blk_seg)],
                sem_o).start()

        for blk in range(_OUT_BLOCKS):
            pltpu.make_async_copy(
                counts_v.at[pl.ds(blk * blk_seg, blk_seg)],
                out_hbm.at[pl.ds(
                    pl.multiple_of(seg_lo + blk * blk_seg, blk_seg),
                    blk_seg)],
                sem_o).wait()
        sc_read.__exit__(None, None, None)

    return k(idx_p, gene_ix, w_flat, b_flat)


def kernel(fragment_coordinates, fragment_cellxgene_ix, fragment_gene_ix,
           cell_n, gene_n, gene_ix, weight1, bias1):
    g_n = gene_ix.shape[0]
    seg_total = _CELL_N * g_n
    idx = fragment_cellxgene_ix.astype(jnp.int32)
    n = idx.shape[0]
    if n % 16:
        n_pad = ((n + 16) // 16) * 16
        idx = jnp.concatenate(
            [idx, jnp.full((n_pad - n,), seg_total, jnp.int32)])
    out_flat = _sc_expression(
        idx,
        gene_ix.astype(jnp.int32),
        weight1.reshape(-1).astype(jnp.float32),
        bias1.astype(jnp.float32),
        seg_total,
    )
    return out_flat.reshape(_CELL_N, g_n)


# speedup vs baseline: 1.0722x; 1.0722x over previous
import functools

import jax
import jax.numpy as jnp
from jax import lax
from jax.experimental import pallas as pl
from jax.experimental.pallas import tpu as pltpu
from jax.experimental.pallas import tpu_sc as plsc

_CELL_N = 16384
_CHUNK = 8192
_CHUNK_SHIFT = 13
_NW = 32
_OUT_BLOCKS = 4


def _sc_expression(idx_p, gene_ix, w_flat, b_flat, seg_total):
    g_n = gene_ix.shape[0]
    seg_per_w = seg_total // _NW
    rows_per_w = seg_per_w // g_n
    blk_rows = rows_per_w // _OUT_BLOCKS
    blk_seg = seg_per_w // _OUT_BLOCKS
    n = idx_p.shape[0]
    n_chunks = -(-n // _CHUNK)
    tail = n - (n_chunks - 1) * _CHUNK
    n_rounds = -(-(n - 1).bit_length() // 4)

    mesh = plsc.VectorSubcoreMesh(core_axis_name="c", subcore_axis_name="s")
    cp = pltpu.CompilerParams(needs_layout_passes=False)

    @functools.partial(
        pl.kernel,
        out_type=jax.ShapeDtypeStruct((seg_total,), jnp.float32),
        mesh=mesh,
        compiler_params=cp,
        scratch_types=[
            pltpu.VMEM((seg_per_w,), jnp.float32),
            pltpu.VMEM((_CHUNK,), jnp.int32),
            pltpu.VMEM((_CHUNK,), jnp.int32),
            pltpu.VMEM((_CHUNK,), jnp.int32),
            pltpu.VMEM((_CHUNK,), jnp.int32),
            pltpu.VMEM((_CHUNK,), jnp.int32),
            pltpu.VMEM((_CHUNK,), jnp.int32),
            pltpu.VMEM((16,), jnp.int32),
            pltpu.VMEM((16,), jnp.int32),
            pltpu.VMEM((16,), jnp.int32),
            pltpu.VMEM((16,), jnp.int32),
            pltpu.VMEM((g_n,), jnp.int32),
            pltpu.VMEM((g_n,), jnp.float32),
            pltpu.VMEM((g_n,), jnp.float32),
            pltpu.VMEM((g_n,), jnp.float32),
            pltpu.VMEM((g_n,), jnp.float32),
            pltpu.SemaphoreType.DMA,
            pltpu.SemaphoreType.DMA,
            pltpu.SemaphoreType.DMA,
            pltpu.SemaphoreType.DMA,
            pltpu.SemaphoreType.DMA,
            pltpu.SemaphoreType.DMA,
            pltpu.SemaphoreType.DMA,
            pltpu.SemaphoreType.DMA,
        ],
    )
    def k(idx_hbm, gix_hbm, w_hbm, b_hbm, out_hbm,
          counts_v, buf_a, buf_b, buf_c, buf_d, buf_e, buf_g,
          probe_a, probe_b, gath_a, gath_b,
          gi_v, wt_v, bt_v, wg_v, bg_v,
          sem_a, sem_b, sem_c, sem_d, sem_e, sem_g, sem_t, sem_o):
        wid = lax.axis_index("s") * 2 + lax.axis_index("c")
        seg_lo = wid * seg_per_w
        seg_hi = seg_lo + seg_per_w

        zeros16 = jnp.zeros((16,), jnp.float32)
        ones16 = jnp.ones((16,), jnp.float32)
        iota16 = lax.iota(jnp.int32, 16)

        pltpu.make_async_copy(gix_hbm, gi_v, sem_t).start()
        pltpu.make_async_copy(w_hbm, wt_v, sem_t).start()
        pltpu.make_async_copy(b_hbm, bt_v, sem_t).start()

        sc_search = jax.named_scope("span_search")
        sc_search.__enter__()
        la = jnp.int32(0)
        lb = jnp.int32(0)
        stage = 16 ** 3
        n_indirect = n_rounds - 3 if n >= stage else n_rounds
        for r in range(n_rounds - 1, n_rounds - 1 - n_indirect, -1):
            step = 16 ** r
            probe_a[...] = jnp.minimum(la + (iota16 + 1) * step - 1, n - 1)
            probe_b[...] = jnp.minimum(lb + (iota16 + 1) * step - 1, n - 1)
            ca = pltpu.async_copy(idx_hbm.at[probe_a], gath_a, sem_a)
            cb = pltpu.async_copy(idx_hbm.at[probe_b], gath_b, sem_b)
            ca.wait()
            cb.wait()
            cnt_a = plsc.all_reduce_population_count(gath_a[...] < seg_lo)
            cnt_b = plsc.all_reduce_population_count(gath_b[...] < seg_hi)
            la = jnp.minimum(la + jnp.max(cnt_a) * step, n)
            lb = jnp.minimum(lb + jnp.max(cnt_b) * step, n)
        if n >= stage:
            base_a = pl.multiple_of(jnp.minimum(la, n - stage), 8)
            base_b = pl.multiple_of(jnp.minimum(lb, n - stage), 8)
            ca = pltpu.async_copy(
                idx_hbm.at[pl.ds(base_a, stage)],
                buf_a.at[pl.ds(0, stage)], sem_a)
            cb = pltpu.async_copy(
                idx_hbm.at[pl.ds(base_b, stage)],
                buf_b.at[pl.ds(0, stage)], sem_b)
            ca.wait()
            cb.wait()
            for r in range(2, -1, -1):
                step = 16 ** r
                pa = jnp.minimum(la + (iota16 + 1) * step - 1, n - 1) - base_a
                pb = jnp.minimum(lb + (iota16 + 1) * step - 1, n - 1) - base_b
                va = plsc.load_gather(buf_a, [pa])
                vb = plsc.load_gather(buf_b, [pb])
                cnt_a = plsc.all_reduce_population_count(va < seg_lo)
                cnt_b = plsc.all_reduce_population_count(vb < seg_hi)
                la = jnp.minimum(la + jnp.max(cnt_a) * step, n)
                lb = jnp.minimum(lb + jnp.max(cnt_b) * step, n)
        frag_lo, frag_hi = la, lb
        sc_search.__exit__(None, None, None)
        sc_zero = jax.named_scope("zero_and_prefetch")
        sc_zero.__enter__()

        c_lo = frag_lo >> _CHUNK_SHIFT
        c_hi = (frag_hi + (_CHUNK - 1)) >> _CHUNK_SHIFT
        nch = c_hi - c_lo
        mid_hi = jnp.maximum(nch - 1, 1)

        def chunk_base(c):
            return pl.multiple_of((c_lo + c) * _CHUNK, _CHUNK)

        def bchunk_copy(glob, base, buf, sem):
            if tail != _CHUNK:
                @pl.when(glob == n_chunks - 1)
                def _():
                    pltpu.make_async_copy(
                        idx_hbm.at[pl.ds(base, tail)],
                        buf.at[pl.ds(0, tail)], sem).start()

                @pl.when(glob != n_chunks - 1)
                def _():
                    pltpu.make_async_copy(
                        idx_hbm.at[pl.ds(base, _CHUNK)], buf, sem).start()
            else:
                pltpu.make_async_copy(
                    idx_hbm.at[pl.ds(base, _CHUNK)], buf, sem).start()

        def bchunk_wait_process(glob, base, buf, sem):
            def masked_subchunks(n_sub):
                @plsc.parallel_loop(0, n_sub, unroll=8)
                def _(j):
                    v = buf[pl.ds(j * 16, 16)]
                    m = (v >= seg_lo) & (v < seg_hi)
                    lv = jnp.where(m, v - seg_lo, 0)
                    plsc.addupdate_scatter(counts_v, [lv], ones16, mask=m)

            if tail != _CHUNK:
                @pl.when(glob == n_chunks - 1)
                def _():
                    pltpu.make_async_copy(
                        idx_hbm.at[pl.ds(base, tail)],
                        buf.at[pl.ds(0, tail)], sem).wait()
                    masked_subchunks(tail // 16)

                @pl.when(glob != n_chunks - 1)
                def _():
                    pltpu.make_async_copy(
                        idx_hbm.at[pl.ds(base, _CHUNK)], buf, sem).wait()
                    masked_subchunks(_CHUNK // 16)
            else:
                pltpu.make_async_copy(
                    idx_hbm.at[pl.ds(base, _CHUNK)], buf, sem).wait()
                masked_subchunks(_CHUNK // 16)

        def start(c, buf, sem):
            pltpu.make_async_copy(
                idx_hbm.at[pl.ds(chunk_base(c), _CHUNK)], buf, sem).start()

        def wait(c, buf, sem):
            pltpu.make_async_copy(
                idx_hbm.at[pl.ds(chunk_base(c), _CHUNK)], buf, sem).wait()

        @pl.when(mid_hi > 1)
        def _():
            start(1, buf_a, sem_a)

        @pl.when(mid_hi > 2)
        def _():
            start(2, buf_b, sem_b)

        @pl.when(nch >= 1)
        def _():
            bchunk_copy(c_lo, chunk_base(0), buf_e, sem_e)

        @pl.when(nch >= 2)
        def _():
            bchunk_copy(c_hi - 1, chunk_base(nch - 1), buf_c, sem_c)

        @plsc.parallel_loop(0, seg_per_w // 16, unroll=8)
        def _(j):
            counts_v[pl.ds(j * 16, 16)] = zeros16

        sc_zero.__exit__(None, None, None)
        sc_hist = jax.named_scope("histogram")
        sc_hist.__enter__()
        @pl.when(nch >= 1)
        def _():
            bchunk_wait_process(c_lo, chunk_base(0), buf_e, sem_e)

        def process(buf):
            @plsc.parallel_loop(0, _CHUNK // 16, unroll=8)
            def _(j):
                v = buf[pl.ds(j * 16, 16)]
                plsc.addupdate_scatter(counts_v, [v - seg_lo], ones16)

        @pl.loop(1, mid_hi, step=4)
        def _(c):
            @pl.when(c + 2 < mid_hi)
            def _():
                start(c + 2, buf_d, sem_d)

            @pl.when(c + 3 < mid_hi)
            def _():
                start(c + 3, buf_g, sem_g)
            wait(c, buf_a, sem_a)
            process(buf_a)

            @pl.when(c + 1 < mid_hi)
            def _():
                wait(c + 1, buf_b, sem_b)
                process(buf_b)

            @pl.when(c + 4 < mid_hi)
            def _():
                start(c + 4, buf_a, sem_a)

            @pl.when(c + 5 < mid_hi)
            def _():
                start(c + 5, buf_b, sem_b)

            @pl.when(c + 2 < mid_hi)
            def _():
                wait(c + 2, buf_d, sem_d)
                process(buf_d)

            @pl.when(c + 3 < mid_hi)
            def _():
                wait(c + 3, buf_g, sem_g)
                process(buf_g)

        @pl.when(nch >= 2)
        def _():
            bchunk_wait_process(c_hi - 1, chunk_base(nch - 1), buf_c, sem_c)

        sc_hist.__exit__(None, None, None)
        sc_read = jax.named_scope("readout")
        sc_read.__enter__()
        pltpu.make_async_copy(gix_hbm, gi_v, sem_t).wait()
        pltpu.make_async_copy(w_hbm, wt_v, sem_t).wait()
        pltpu.make_async_copy(b_hbm, bt_v, sem_t).wait()
        for t in range(g_n // 16):
            g16 = gi_v[pl.ds(t * 16, 16)]
            wg_v[pl.ds(t * 16, 16)] = plsc.load_gather(wt_v, [g16])
            bg_v[pl.ds(t * 16, 16)] = plsc.load_gather(bt_v, [g16])

        for blk in range(_OUT_BLOCKS):
            @plsc.parallel_loop(blk * blk_rows, (blk + 1) * blk_rows,
                                unroll=8)
            def _(r):
                row = r * g_n
                for t in range(g_n // 16):
                    off = row + t * 16
                    counts_v[pl.ds(off, 16)] = (
                        counts_v[pl.ds(off, 16)] * wg_v[pl.ds(t * 16, 16)]
                        + bg_v[pl.ds(t * 16, 16)])
            pltpu.make_async_copy(
                counts_v.at[pl.ds(blk * blk_seg, blk_seg)],
                out_hbm.at[pl.ds(
                    pl.multiple_of(seg_lo + blk * blk_seg, blk_seg),
                    blk_seg)],
                sem_o).start()

        for blk in range(_OUT_BLOCKS):
            pltpu.make_async_copy(
                counts_v.at[pl.ds(blk * blk_seg, blk_seg)],
                out_hbm.at[pl.ds(
                    pl.multiple_of(seg_lo + blk * blk_seg, blk_seg),
                    blk_seg)],
                sem_o).wait()
        sc_read.__exit__(None, None, None)

    return k(idx_p, gene_ix, w_flat, b_flat)


def kernel(fragment_coordinates, fragment_cellxgene_ix, fragment_gene_ix,
           cell_n, gene_n, gene_ix, weight1, bias1):
    g_n = gene_ix.shape[0]
    seg_total = _CELL_N * g_n
    idx = fragment_cellxgene_ix.astype(jnp.int32)
    n = idx.shape[0]
    if n % 16:
        n_pad = ((n + 16) // 16) * 16
        idx = jnp.concatenate(
            [idx, jnp.full((n_pad - n,), seg_total, jnp.int32)])
    out_flat = _sc_expression(
        idx,
        gene_ix.astype(jnp.int32),
        weight1.reshape(-1).astype(jnp.float32),
        bias1.astype(jnp.float32),
        seg_total,
    )
    return out_flat.reshape(_CELL_N, g_n)


# speedup vs baseline: 1.0781x; 1.0055x over previous
import functools

import jax
import jax.numpy as jnp
from jax import lax
from jax.experimental import pallas as pl
from jax.experimental.pallas import tpu as pltpu
from jax.experimental.pallas import tpu_sc as plsc

_CELL_N = 16384
_CHUNK = 8192
_CHUNK_SHIFT = 13
_NW = 32
_OUT_BLOCKS = 4


def _sc_expression(idx_p, gene_ix, w_flat, b_flat, seg_total):
    g_n = gene_ix.shape[0]
    seg_per_w = seg_total // _NW
    rows_per_w = seg_per_w // g_n
    blk_rows = rows_per_w // _OUT_BLOCKS
    blk_seg = seg_per_w // _OUT_BLOCKS
    n = idx_p.shape[0]
    n_chunks = -(-n // _CHUNK)
    tail = n - (n_chunks - 1) * _CHUNK
    n_rounds = -(-(n - 1).bit_length() // 4)

    mesh = plsc.VectorSubcoreMesh(core_axis_name="c", subcore_axis_name="s")
    cp = pltpu.CompilerParams(needs_layout_passes=False)

    @functools.partial(
        pl.kernel,
        out_type=jax.ShapeDtypeStruct((seg_total,), jnp.float32),
        mesh=mesh,
        compiler_params=cp,
        scratch_types=[
            pltpu.VMEM((seg_per_w,), jnp.float32),
            pltpu.VMEM((_CHUNK,), jnp.int32),
            pltpu.VMEM((_CHUNK,), jnp.int32),
            pltpu.VMEM((_CHUNK,), jnp.int32),
            pltpu.VMEM((_CHUNK,), jnp.int32),
            pltpu.VMEM((16,), jnp.int32),
            pltpu.VMEM((16,), jnp.int32),
            pltpu.VMEM((16,), jnp.int32),
            pltpu.VMEM((16,), jnp.int32),
            pltpu.VMEM((g_n,), jnp.int32),
            pltpu.VMEM((g_n,), jnp.float32),
            pltpu.VMEM((g_n,), jnp.float32),
            pltpu.VMEM((g_n,), jnp.float32),
            pltpu.VMEM((g_n,), jnp.float32),
            pltpu.SemaphoreType.DMA,
            pltpu.SemaphoreType.DMA,
            pltpu.SemaphoreType.DMA,
            pltpu.SemaphoreType.DMA,
            pltpu.SemaphoreType.DMA,
            pltpu.SemaphoreType.DMA,
        ],
    )
    def k(idx_hbm, gix_hbm, w_hbm, b_hbm, out_hbm,
          counts_v, buf_a, buf_b, buf_c, buf_e,
          probe_a, probe_b, gath_a, gath_b,
          gi_v, wt_v, bt_v, wg_v, bg_v,
          sem_a, sem_b, sem_c, sem_e, sem_t, sem_o):
        wid = lax.axis_index("s") * 2 + lax.axis_index("c")
        seg_lo = wid * seg_per_w
        seg_hi = seg_lo + seg_per_w

        zeros16 = jnp.zeros((16,), jnp.float32)
        ones16 = jnp.ones((16,), jnp.float32)
        iota16 = lax.iota(jnp.int32, 16)

        pltpu.make_async_copy(gix_hbm, gi_v, sem_t).start()
        pltpu.make_async_copy(w_hbm, wt_v, sem_t).start()
        pltpu.make_async_copy(b_hbm, bt_v, sem_t).start()

        sc_search = jax.named_scope("span_search")
        sc_search.__enter__()
        la = jnp.int32(0)
        lb = jnp.int32(0)
        stage = 16 ** 3
        n_indirect = n_rounds - 3 if n >= stage else n_rounds
        for r in range(n_rounds - 1, n_rounds - 1 - n_indirect, -1):
            step = 16 ** r
            probe_a[...] = jnp.minimum(la + (iota16 + 1) * step - 1, n - 1)
            probe_b[...] = jnp.minimum(lb + (iota16 + 1) * step - 1, n - 1)
            ca = pltpu.async_copy(idx_hbm.at[probe_a], gath_a, sem_a)
            cb = pltpu.async_copy(idx_hbm.at[probe_b], gath_b, sem_b)
            ca.wait()
            cb.wait()
            cnt_a = plsc.all_reduce_population_count(gath_a[...] < seg_lo)
            cnt_b = plsc.all_reduce_population_count(gath_b[...] < seg_hi)
            la = jnp.minimum(la + jnp.max(cnt_a) * step, n)
            lb = jnp.minimum(lb + jnp.max(cnt_b) * step, n)
        if n >= stage:
            base_a = pl.multiple_of(jnp.minimum(la, n - stage), 8)
            base_b = pl.multiple_of(jnp.minimum(lb, n - stage), 8)
            ca = pltpu.async_copy(
                idx_hbm.at[pl.ds(base_a, stage)],
                buf_a.at[pl.ds(0, stage)], sem_a)
            cb = pltpu.async_copy(
                idx_hbm.at[pl.ds(base_b, stage)],
                buf_b.at[pl.ds(0, stage)], sem_b)
            ca.wait()
            cb.wait()
            for r in range(2, -1, -1):
                step = 16 ** r
                pa = jnp.minimum(la + (iota16 + 1) * step - 1, n - 1) - base_a
                pb = jnp.minimum(lb + (iota16 + 1) * step - 1, n - 1) - base_b
                va = plsc.load_gather(buf_a, [pa])
                vb = plsc.load_gather(buf_b, [pb])
                cnt_a = plsc.all_reduce_population_count(va < seg_lo)
                cnt_b = plsc.all_reduce_population_count(vb < seg_hi)
                la = jnp.minimum(la + jnp.max(cnt_a) * step, n)
                lb = jnp.minimum(lb + jnp.max(cnt_b) * step, n)
        frag_lo, frag_hi = la, lb
        sc_search.__exit__(None, None, None)
        sc_zero = jax.named_scope("zero_and_prefetch")
        sc_zero.__enter__()

        c_lo = frag_lo >> _CHUNK_SHIFT
        c_hi = (frag_hi + (_CHUNK - 1)) >> _CHUNK_SHIFT
        nch = c_hi - c_lo
        mid_hi = jnp.maximum(nch - 1, 1)

        def chunk_base(c):
            return pl.multiple_of((c_lo + c) * _CHUNK, _CHUNK)

        def bchunk_copy(glob, base, buf, sem):
            if tail != _CHUNK:
                @pl.when(glob == n_chunks - 1)
                def _():
                    pltpu.make_async_copy(
                        idx_hbm.at[pl.ds(base, tail)],
                        buf.at[pl.ds(0, tail)], sem).start()

                @pl.when(glob != n_chunks - 1)
                def _():
                    pltpu.make_async_copy(
                        idx_hbm.at[pl.ds(base, _CHUNK)], buf, sem).start()
            else:
                pltpu.make_async_copy(
                    idx_hbm.at[pl.ds(base, _CHUNK)], buf, sem).start()

        def bchunk_wait_process(glob, base, buf, sem):
            def masked_subchunks(n_sub):
                @plsc.parallel_loop(0, n_sub, unroll=8)
                def _(j):
                    v = buf[pl.ds(j * 16, 16)]
                    m = (v >= seg_lo) & (v < seg_hi)
                    lv = jnp.where(m, v - seg_lo, 0)
                    plsc.addupdate_scatter(counts_v, [lv], ones16, mask=m)

            if tail != _CHUNK:
                @pl.when(glob == n_chunks - 1)
                def _():
                    pltpu.make_async_copy(
                        idx_hbm.at[pl.ds(base, tail)],
                        buf.at[pl.ds(0, tail)], sem).wait()
                    masked_subchunks(tail // 16)

                @pl.when(glob != n_chunks - 1)
                def _():
                    pltpu.make_async_copy(
                        idx_hbm.at[pl.ds(base, _CHUNK)], buf, sem).wait()
                    masked_subchunks(_CHUNK // 16)
            else:
                pltpu.make_async_copy(
                    idx_hbm.at[pl.ds(base, _CHUNK)], buf, sem).wait()
                masked_subchunks(_CHUNK // 16)

        def start(c, buf, sem):
            pltpu.make_async_copy(
                idx_hbm.at[pl.ds(chunk_base(c), _CHUNK)], buf, sem).start()

        def wait(c, buf, sem):
            pltpu.make_async_copy(
                idx_hbm.at[pl.ds(chunk_base(c), _CHUNK)], buf, sem).wait()

        @pl.when(mid_hi > 1)
        def _():
            start(1, buf_a, sem_a)

        @pl.when(nch >= 1)
        def _():
            bchunk_copy(c_lo, chunk_base(0), buf_e, sem_e)

        @pl.when(nch >= 2)
        def _():
            bchunk_copy(c_hi - 1, chunk_base(nch - 1), buf_c, sem_c)

        @plsc.parallel_loop(0, seg_per_w // 16, unroll=8)
        def _(j):
            counts_v[pl.ds(j * 16, 16)] = zeros16

        sc_zero.__exit__(None, None, None)
        sc_hist = jax.named_scope("histogram")
        sc_hist.__enter__()
        @pl.when(nch >= 1)
        def _():
            bchunk_wait_process(c_lo, chunk_base(0), buf_e, sem_e)

        def process(buf):
            @plsc.parallel_loop(0, _CHUNK // 16, unroll=8)
            def _(j):
                v = buf[pl.ds(j * 16, 16)]
                plsc.addupdate_scatter(counts_v, [v - seg_lo], ones16)

        @pl.loop(1, mid_hi, step=2)
        def _(c):
            @pl.when(c + 1 < mid_hi)
            def _():
                start(c + 1, buf_b, sem_b)
            wait(c, buf_a, sem_a)
            process(buf_a)

            @pl.when(c + 1 < mid_hi)
            def _():
                @pl.when(c + 2 < mid_hi)
                def _():
                    start(c + 2, buf_a, sem_a)
                wait(c + 1, buf_b, sem_b)
                process(buf_b)

        @pl.when(nch >= 2)
        def _():
            bchunk_wait_process(c_hi - 1, chunk_base(nch - 1), buf_c, sem_c)

        sc_hist.__exit__(None, None, None)
        sc_read = jax.named_scope("readout")
        sc_read.__enter__()
        pltpu.make_async_copy(gix_hbm, gi_v, sem_t).wait()
        pltpu.make_async_copy(w_hbm, wt_v, sem_t).wait()
        pltpu.make_async_copy(b_hbm, bt_v, sem_t).wait()
        for t in range(g_n // 16):
            g16 = gi_v[pl.ds(t * 16, 16)]
            wg_v[pl.ds(t * 16, 16)] = plsc.load_gather(wt_v, [g16])
            bg_v[pl.ds(t * 16, 16)] = plsc.load_gather(bt_v, [g16])

        for blk in range(_OUT_BLOCKS):
            @plsc.parallel_loop(blk * blk_rows, (blk + 1) * blk_rows,
                                unroll=8)
            def _(r):
                row = r * g_n
                for t in range(g_n // 16):
                    off = row + t * 16
                    counts_v[pl.ds(off, 16)] = (
                        counts_v[pl.ds(off, 16)] * wg_v[pl.ds(t * 16, 16)]
                        + bg_v[pl.ds(t * 16, 16)])
            pltpu.make_async_copy(
                counts_v.at[pl.ds(blk * blk_seg, blk_seg)],
                out_hbm.at[pl.ds(
                    pl.multiple_of(seg_lo + blk * blk_seg, blk_seg),
                    blk_seg)],
                sem_o).start()

        for blk in range(_OUT_BLOCKS):
            pltpu.make_async_copy(
                counts_v.at[pl.ds(blk * blk_seg, blk_seg)],
                out_hbm.at[pl.ds(
                    pl.multiple_of(seg_lo + blk * blk_seg, blk_seg),
                    blk_seg)],
                sem_o).wait()
        sc_read.__exit__(None, None, None)

    return k(idx_p, gene_ix, w_flat, b_flat)


def kernel(fragment_coordinates, fragment_cellxgene_ix, fragment_gene_ix,
           cell_n, gene_n, gene_ix, weight1, bias1):
    g_n = gene_ix.shape[0]
    seg_total = _CELL_N * g_n
    idx = fragment_cellxgene_ix.astype(jnp.int32)
    n = idx.shape[0]
    if n % 16:
        n_pad = ((n + 16) // 16) * 16
        idx = jnp.concatenate(
            [idx, jnp.full((n_pad - n,), seg_total, jnp.int32)])
    out_flat = _sc_expression(
        idx,
        gene_ix.astype(jnp.int32),
        weight1.reshape(-1).astype(jnp.float32),
        bias1.astype(jnp.float32),
        seg_total,
    )
    return out_flat.reshape(_CELL_N, g_n)


# speedup vs baseline: 1.1016x; 1.0218x over previous
import functools

import jax
import jax.numpy as jnp
from jax import lax
from jax.experimental import pallas as pl
from jax.experimental.pallas import tpu as pltpu
from jax.experimental.pallas import tpu_sc as plsc

_CELL_N = 16384
_CHUNK = 8192
_CHUNK_SHIFT = 13
_NW = 32
_OUT_BLOCKS = 2


def _sc_expression(idx_p, gene_ix, w_flat, b_flat, seg_total):
    g_n = gene_ix.shape[0]
    seg_per_w = seg_total // _NW
    rows_per_w = seg_per_w // g_n
    blk_rows = rows_per_w // _OUT_BLOCKS
    blk_seg = seg_per_w // _OUT_BLOCKS
    n = idx_p.shape[0]
    n_chunks = -(-n // _CHUNK)
    tail = n - (n_chunks - 1) * _CHUNK
    n_rounds = -(-(n - 1).bit_length() // 4)

    mesh = plsc.VectorSubcoreMesh(core_axis_name="c", subcore_axis_name="s")
    cp = pltpu.CompilerParams(needs_layout_passes=False)

    @functools.partial(
        pl.kernel,
        out_type=jax.ShapeDtypeStruct((seg_total,), jnp.float32),
        mesh=mesh,
        compiler_params=cp,
        scratch_types=[
            pltpu.VMEM((seg_per_w,), jnp.float32),
            pltpu.VMEM((_CHUNK,), jnp.int32),
            pltpu.VMEM((_CHUNK,), jnp.int32),
            pltpu.VMEM((_CHUNK,), jnp.int32),
            pltpu.VMEM((_CHUNK,), jnp.int32),
            pltpu.VMEM((16,), jnp.int32),
            pltpu.VMEM((16,), jnp.int32),
            pltpu.VMEM((16,), jnp.int32),
            pltpu.VMEM((16,), jnp.int32),
            pltpu.VMEM((g_n,), jnp.int32),
            pltpu.VMEM((g_n,), jnp.float32),
            pltpu.VMEM((g_n,), jnp.float32),
            pltpu.VMEM((g_n,), jnp.float32),
            pltpu.VMEM((g_n,), jnp.float32),
            pltpu.SemaphoreType.DMA,
            pltpu.SemaphoreType.DMA,
            pltpu.SemaphoreType.DMA,
            pltpu.SemaphoreType.DMA,
            pltpu.SemaphoreType.DMA,
            pltpu.SemaphoreType.DMA,
        ],
    )
    def k(idx_hbm, gix_hbm, w_hbm, b_hbm, out_hbm,
          counts_v, buf_a, buf_b, buf_c, buf_e,
          probe_a, probe_b, gath_a, gath_b,
          gi_v, wt_v, bt_v, wg_v, bg_v,
          sem_a, sem_b, sem_c, sem_e, sem_t, sem_o):
        wid = lax.axis_index("s") * 2 + lax.axis_index("c")
        seg_lo = wid * seg_per_w
        seg_hi = seg_lo + seg_per_w

        zeros16 = jnp.zeros((16,), jnp.float32)
        ones16 = jnp.ones((16,), jnp.float32)
        iota16 = lax.iota(jnp.int32, 16)

        pltpu.make_async_copy(gix_hbm, gi_v, sem_t).start()
        pltpu.make_async_copy(w_hbm, wt_v, sem_t).start()
        pltpu.make_async_copy(b_hbm, bt_v, sem_t).start()

        sc_search = jax.named_scope("span_search")
        sc_search.__enter__()
        la = jnp.int32(0)
        lb = jnp.int32(0)
        stage = 16 ** 3
        n_indirect = n_rounds - 3 if n >= stage else n_rounds
        for r in range(n_rounds - 1, n_rounds - 1 - n_indirect, -1):
            step = 16 ** r
            probe_a[...] = jnp.minimum(la + (iota16 + 1) * step - 1, n - 1)
            probe_b[...] = jnp.minimum(lb + (iota16 + 1) * step - 1, n - 1)
            ca = pltpu.async_copy(idx_hbm.at[probe_a], gath_a, sem_a)
            cb = pltpu.async_copy(idx_hbm.at[probe_b], gath_b, sem_b)
            ca.wait()
            cb.wait()
            cnt_a = plsc.all_reduce_population_count(gath_a[...] < seg_lo)
            cnt_b = plsc.all_reduce_population_count(gath_b[...] < seg_hi)
            la = jnp.minimum(la + jnp.max(cnt_a) * step, n)
            lb = jnp.minimum(lb + jnp.max(cnt_b) * step, n)
        if n >= stage:
            base_a = pl.multiple_of(jnp.minimum(la, n - stage), 8)
            base_b = pl.multiple_of(jnp.minimum(lb, n - stage), 8)
            ca = pltpu.async_copy(
                idx_hbm.at[pl.ds(base_a, stage)],
                buf_a.at[pl.ds(0, stage)], sem_a)
            cb = pltpu.async_copy(
                idx_hbm.at[pl.ds(base_b, stage)],
                buf_b.at[pl.ds(0, stage)], sem_b)
            ca.wait()
            cb.wait()
            for r in range(2, -1, -1):
                step = 16 ** r
                pa = jnp.minimum(la + (iota16 + 1) * step - 1, n - 1) - base_a
                pb = jnp.minimum(lb + (iota16 + 1) * step - 1, n - 1) - base_b
                va = plsc.load_gather(buf_a, [pa])
                vb = plsc.load_gather(buf_b, [pb])
                cnt_a = plsc.all_reduce_population_count(va < seg_lo)
                cnt_b = plsc.all_reduce_population_count(vb < seg_hi)
                la = jnp.minimum(la + jnp.max(cnt_a) * step, n)
                lb = jnp.minimum(lb + jnp.max(cnt_b) * step, n)
        frag_lo, frag_hi = la, lb
        sc_search.__exit__(None, None, None)
        sc_zero = jax.named_scope("zero_and_prefetch")
        sc_zero.__enter__()

        c_lo = frag_lo >> _CHUNK_SHIFT
        c_hi = (frag_hi + (_CHUNK - 1)) >> _CHUNK_SHIFT
        nch = c_hi - c_lo
        mid_hi = jnp.maximum(nch - 1, 1)

        def chunk_base(c):
            return pl.multiple_of((c_lo + c) * _CHUNK, _CHUNK)

        def bchunk_copy(glob, base, buf, sem):
            if tail != _CHUNK:
                @pl.when(glob == n_chunks - 1)
                def _():
                    pltpu.make_async_copy(
                        idx_hbm.at[pl.ds(base, tail)],
                        buf.at[pl.ds(0, tail)], sem).start()

                @pl.when(glob != n_chunks - 1)
                def _():
                    pltpu.make_async_copy(
                        idx_hbm.at[pl.ds(base, _CHUNK)], buf, sem).start()
            else:
                pltpu.make_async_copy(
                    idx_hbm.at[pl.ds(base, _CHUNK)], buf, sem).start()

        def bchunk_wait_process(glob, base, buf, sem):
            def masked_subchunks(n_sub):
                @plsc.parallel_loop(0, n_sub, unroll=8)
                def _(j):
                    v = buf[pl.ds(j * 16, 16)]
                    m = (v >= seg_lo) & (v < seg_hi)
                    lv = jnp.where(m, v - seg_lo, 0)
                    plsc.addupdate_scatter(counts_v, [lv], ones16, mask=m)

            if tail != _CHUNK:
                @pl.when(glob == n_chunks - 1)
                def _():
                    pltpu.make_async_copy(
                        idx_hbm.at[pl.ds(base, tail)],
                        buf.at[pl.ds(0, tail)], sem).wait()
                    masked_subchunks(tail // 16)

                @pl.when(glob != n_chunks - 1)
                def _():
                    pltpu.make_async_copy(
                        idx_hbm.at[pl.ds(base, _CHUNK)], buf, sem).wait()
                    masked_subchunks(_CHUNK // 16)
            else:
                pltpu.make_async_copy(
                    idx_hbm.at[pl.ds(base, _CHUNK)], buf, sem).wait()
                masked_subchunks(_CHUNK // 16)

        def start(c, buf, sem):
            pltpu.make_async_copy(
                idx_hbm.at[pl.ds(chunk_base(c), _CHUNK)], buf, sem).start()

        def wait(c, buf, sem):
            pltpu.make_async_copy(
                idx_hbm.at[pl.ds(chunk_base(c), _CHUNK)], buf, sem).wait()

        @pl.when(mid_hi > 1)
        def _():
            start(1, buf_a, sem_a)

        @pl.when(nch >= 1)
        def _():
            bchunk_copy(c_lo, chunk_base(0), buf_e, sem_e)

        @pl.when(nch >= 2)
        def _():
            bchunk_copy(c_hi - 1, chunk_base(nch - 1), buf_c, sem_c)

        @plsc.parallel_loop(0, seg_per_w // 16, unroll=8)
        def _(j):
            counts_v[pl.ds(j * 16, 16)] = zeros16

        sc_zero.__exit__(None, None, None)
        sc_hist = jax.named_scope("histogram")
        sc_hist.__enter__()
        @pl.when(nch >= 1)
        def _():
            bchunk_wait_process(c_lo, chunk_base(0), buf_e, sem_e)

        def process(buf):
            @plsc.parallel_loop(0, _CHUNK // 16, unroll=8)
            def _(j):
                v = buf[pl.ds(j * 16, 16)]
                plsc.addupdate_scatter(counts_v, [v - seg_lo], ones16)

        @pl.loop(1, mid_hi, step=2)
        def _(c):
            @pl.when(c + 1 < mid_hi)
            def _():
                start(c + 1, buf_b, sem_b)
            wait(c, buf_a, sem_a)
            process(buf_a)

            @pl.when(c + 1 < mid_hi)
            def _():
                @pl.when(c + 2 < mid_hi)
                def _():
                    start(c + 2, buf_a, sem_a)
                wait(c + 1, buf_b, sem_b)
                process(buf_b)

        @pl.when(nch >= 2)
        def _():
            bchunk_wait_process(c_hi - 1, chunk_base(nch - 1), buf_c, sem_c)

        sc_hist.__exit__(None, None, None)
        sc_read = jax.named_scope("readout")
        sc_read.__enter__()
        pltpu.make_async_copy(gix_hbm, gi_v, sem_t).wait()
        pltpu.make_async_copy(w_hbm, wt_v, sem_t).wait()
        pltpu.make_async_copy(b_hbm, bt_v, sem_t).wait()
        for t in range(g_n // 16):
            g16 = gi_v[pl.ds(t * 16, 16)]
            wg_v[pl.ds(t * 16, 16)] = plsc.load_gather(wt_v, [g16])
            bg_v[pl.ds(t * 16, 16)] = plsc.load_gather(bt_v, [g16])

        for blk in range(_OUT_BLOCKS):
            @plsc.parallel_loop(blk * blk_rows, (blk + 1) * blk_rows,
                                unroll=8)
            def _(r):
                row = r * g_n
                for t in range(g_n // 16):
                    off = row + t * 16
                    counts_v[pl.ds(off, 16)] = (
                        counts_v[pl.ds(off, 16)] * wg_v[pl.ds(t * 16, 16)]
                        + bg_v[pl.ds(t * 16, 16)])
            pltpu.make_async_copy(
                counts_v.at[pl.ds(blk * blk_seg, blk_seg)],
                out_hbm.at[pl.ds(
                    pl.multiple_of(seg_lo + blk * blk_seg, blk_seg),
                    blk_seg)],
                sem_o).start()

        for blk in range(_OUT_BLOCKS):
            pltpu.make_async_copy(
                counts_v.at[pl.ds(blk * blk_seg, blk_seg)],
                out_hbm.at[pl.ds(
                    pl.multiple_of(seg_lo + blk * blk_seg, blk_seg),
                    blk_seg)],
                sem_o).wait()
        sc_read.__exit__(None, None, None)

    return k(idx_p, gene_ix, w_flat, b_flat)


def kernel(fragment_coordinates, fragment_cellxgene_ix, fragment_gene_ix,
           cell_n, gene_n, gene_ix, weight1, bias1):
    g_n = gene_ix.shape[0]
    seg_total = _CELL_N * g_n
    idx = fragment_cellxgene_ix.astype(jnp.int32)
    n = idx.shape[0]
    if n % 16:
        n_pad = ((n + 16) // 16) * 16
        idx = jnp.concatenate(
            [idx, jnp.full((n_pad - n,), seg_total, jnp.int32)])
    out_flat = _sc_expression(
        idx,
        gene_ix.astype(jnp.int32),
        weight1.reshape(-1).astype(jnp.float32),
        bias1.astype(jnp.float32),
        seg_total,
    )
    return out_flat.reshape(_CELL_N, g_n)
